# baseline (speedup 1.0000x reference)
import functools

import jax
import jax.numpy as jnp
from jax import lax
from jax.experimental import pallas as pl
from jax.experimental.pallas import tpu as pltpu
from jax.experimental.pallas import tpu_sc as plsc

N_NODES = 10000
N_EDGES = 160000
D = 256
H = 128
NS = 16
K = 64
NBUF = 2
NSTAGE = 4
CH = 40
NCHUNK = NSTAGE * CH
NP = 10240
NR = NP // 2
RPT = NR // NS
NTRASH = 128
ACC_R = NR + NTRASH
E_PAD = NS * NCHUNK * K
EPT = NCHUNK * K


def _aggr_core(tab, out, base, s, srcs_t, dsts_t, src_v, dst_v, rows_v, acc,
               gsem, ssem):
    pltpu.sync_copy(tab.at[pl.ds(base + s * RPT, RPT)],
                    acc.at[pl.ds(s * RPT, RPT)])
    plsc.subcore_barrier()

    for st in range(NSTAGE):
        pltpu.sync_copy(srcs_t.at[st], src_v)
        pltpu.sync_copy(dsts_t.at[st], dst_v)

        for m in range(NBUF - 1):
            pltpu.async_copy(tab.at[src_v.at[m]], rows_v.at[m], gsem.at[m])

        @pl.loop(0, CH, step=NBUF)
        def _group(j):
            for b in range(NBUF):
                cur = j + b
                pre = cur + NBUF - 1
                pb = (cur + NBUF - 1) % NBUF

                @pl.when(cur > 0)
                def _():
                    pltpu.make_async_copy(rows_v.at[pb],
                                          acc.at[dst_v.at[cur - 1]],
                                          ssem.at[pb]).wait()

                @pl.when(pre < CH)
                def _():
                    pltpu.async_copy(tab.at[src_v.at[pre]], rows_v.at[pb],
                                     gsem.at[pb])

                pltpu.make_async_copy(tab.at[src_v.at[cur]], rows_v.at[b],
                                      gsem.at[b]).wait()
                pltpu.async_copy(rows_v.at[b], acc.at[dst_v.at[cur]],
                                 ssem.at[b], add=True)

        lb = (CH - 1) % NBUF
        pltpu.make_async_copy(rows_v.at[lb], acc.at[dst_v.at[CH - 1]],
                              ssem.at[lb]).wait()

    plsc.subcore_barrier()
    pltpu.sync_copy(acc.at[pl.ds(s * RPT, RPT)],
                    out.at[pl.ds(base + s * RPT, RPT)])


@functools.cache
def _make_sc_aggr():
    @functools.partial(
        pl.kernel,
        out_type=jax.ShapeDtypeStruct((NP, D), jnp.float32),
        mesh=plsc.VectorSubcoreMesh(core_axis_name="c", subcore_axis_name="s"),
        compiler_params=pltpu.CompilerParams(use_tc_tiling_on_sc=False),
        scratch_types=[
            pltpu.VMEM((CH, K), jnp.int32),
            pltpu.VMEM((CH, K), jnp.int32),
            pltpu.VMEM((NBUF, K, D), jnp.float32),
            pltpu.VMEM_SHARED((ACC_R, D), jnp.float32),
            pltpu.SemaphoreType.DMA((NBUF,)),
            pltpu.SemaphoreType.DMA((NBUF,)),
        ],
    )
    def _sc_aggr(tab, srcs, dsts, out, src_v, dst_v, rows_v, acc, gsem, ssem):
        s = lax.axis_index("s")
        c = lax.axis_index("c")

        @pl.when(c == 0)
        def _():
            _aggr_core(tab, out, 0, s, srcs.at[s], dsts.at[0].at[s],
                       src_v, dst_v, rows_v, acc, gsem, ssem)

        @pl.when(c == 1)
        def _():
            _aggr_core(tab, out, NR, s, srcs.at[s], dsts.at[1].at[s],
                       src_v, dst_v, rows_v, acc, gsem, ssem)

    return _sc_aggr


def _mlp_body(relu_out, z_ref, wa_ref, ba_ref, wb_ref, bb_ref, out_ref):
    h = jnp.dot(z_ref[...], wa_ref[...], preferred_element_type=jnp.float32)
    h = jnp.maximum(h + ba_ref[...], 0.0)
    o = jnp.dot(h, wb_ref[...], preferred_element_type=jnp.float32) + bb_ref[...]
    if relu_out:
        o = jnp.maximum(o, 0.0)
    out_ref[...] = o


def _mlp(z, wa, ba, wb, bb, relu_out, n_rows, blk):
    return pl.pallas_call(
        functools.partial(_mlp_body, relu_out),
        grid=(n_rows // blk,),
        in_specs=[
            pl.BlockSpec((blk, D), lambda i: (i, 0)),
            pl.BlockSpec((D, D), lambda i: (0, 0)),
            pl.BlockSpec((1, D), lambda i: (0, 0)),
            pl.BlockSpec((D, D), lambda i: (0, 0)),
            pl.BlockSpec((1, D), lambda i: (0, 0)),
        ],
        out_specs=pl.BlockSpec((blk, D), lambda i: (i, 0)),
        out_shape=jax.ShapeDtypeStruct((n_rows, D), jnp.float32),
    )(z, wa, ba.reshape(1, D), wb, bb.reshape(1, D))


def kernel(x, edge_index, W1a, b1a, W1b, b1b, W2a, b2a, W2b, b2b):
    pad_e = E_PAD - N_EDGES
    src = jnp.concatenate([edge_index[0], jnp.zeros((pad_e,), jnp.int32)])
    dst = jnp.concatenate([edge_index[1], jnp.full((pad_e,), -1, jnp.int32)])
    eidx = jnp.arange(E_PAD, dtype=jnp.int32)
    trash = NR + 8 * (eidx // EPT) + (eidx % 8)
    dloc = []
    for c in range(2):
        lo = c * NR
        inr = (dst >= lo) & (dst < lo + NR)
        dloc.append(jnp.where(inr, dst - lo, trash))
    srcs = src.reshape(NS, NSTAGE, CH, K)
    dsts = jnp.stack(dloc).reshape(2, NS, NSTAGE, CH, K)
    xp = jnp.pad(x, ((0, NP - N_NODES), (0, 0)))
    sc_aggr = _make_sc_aggr()
    z1 = sc_aggr(xp, srcs, dsts)
    h = _mlp(z1, W1a, b1a, W1b, b1b, relu_out=True, n_rows=NP, blk=2048)
    z2 = sc_aggr(h, srcs, dsts)
    return _mlp(z2, W2a, b2a, W2b, b2b, relu_out=False, n_rows=N_NODES,
                blk=2000)


# speedup vs baseline: 1.7029x; 1.7029x over previous
import functools

import jax
import jax.numpy as jnp
from jax import lax
from jax.experimental import pallas as pl
from jax.experimental.pallas import tpu as pltpu
from jax.experimental.pallas import tpu_sc as plsc

N_NODES = 10000
N_EDGES = 160000
D = 256
H = 128
NS = 16
K = 64
NBUF = 4
NSTAGE = 4
CH = 40
NCHUNK = NSTAGE * CH
NP = 10240
RPT = NP // NS
E_PAD = NS * NCHUNK * K


def _aggr_half(tab, out, s, srcs_t, dsts_t, src_v, dst_v, rows_v, acc, gsem,
               ssem):
    off = s * RPT
    pltpu.sync_copy(tab.at[pl.ds(off, RPT)], acc.at[pl.ds(off, RPT)])
    plsc.subcore_barrier()

    for st in range(NSTAGE):
        pltpu.sync_copy(srcs_t.at[st], src_v)
        pltpu.sync_copy(dsts_t.at[st], dst_v)

        for m in range(NBUF - 1):
            pltpu.async_copy(tab.at[src_v.at[m]], rows_v.at[m], gsem.at[m])

        @pl.loop(0, CH, step=NBUF)
        def _group(j):
            for b in range(NBUF):
                cur = j + b
                pre = cur + NBUF - 1
                pb = (cur + NBUF - 1) % NBUF

                @pl.when(cur > 0)
                def _():
                    pltpu.make_async_copy(rows_v.at[pb],
                                          acc.at[dst_v.at[cur - 1]],
                                          ssem.at[pb]).wait()

                @pl.when(pre < CH)
                def _():
                    pltpu.async_copy(tab.at[src_v.at[pre]], rows_v.at[pb],
                                     gsem.at[pb])

                pltpu.make_async_copy(tab.at[src_v.at[cur]], rows_v.at[b],
                                      gsem.at[b]).wait()
                pltpu.async_copy(rows_v.at[b], acc.at[dst_v.at[cur]],
                                 ssem.at[b], add=True)

        pltpu.make_async_copy(rows_v.at[(CH - 1) % NBUF],
                              acc.at[dst_v.at[CH - 1]],
                              ssem.at[(CH - 1) % NBUF]).wait()

    plsc.subcore_barrier()
    pltpu.sync_copy(acc.at[pl.ds(off, RPT)], out.at[pl.ds(off, RPT)])


@functools.cache
def _make_sc_aggr():
    @functools.partial(
        pl.kernel,
        out_type=jax.ShapeDtypeStruct((2, NP, H), jnp.float32),
        mesh=plsc.VectorSubcoreMesh(core_axis_name="c", subcore_axis_name="s"),
        scratch_types=[
            pltpu.VMEM((CH, K), jnp.int32),
            pltpu.VMEM((CH, K), jnp.int32),
            pltpu.VMEM((NBUF, K, H), jnp.float32),
            pltpu.VMEM_SHARED((NP, H), jnp.float32),
            pltpu.SemaphoreType.DMA((NBUF,)),
            pltpu.SemaphoreType.DMA((NBUF,)),
        ],
    )
    def _sc_aggr(tab, srcs, dsts, out, src_v, dst_v, rows_v, acc, gsem, ssem):
        s = lax.axis_index("s")
        c = lax.axis_index("c")

        @pl.when(c == 0)
        def _():
            _aggr_half(tab.at[0], out.at[0], s, srcs.at[s], dsts.at[s],
                       src_v, dst_v, rows_v, acc, gsem, ssem)

        @pl.when(c == 1)
        def _():
            _aggr_half(tab.at[1], out.at[1], s, srcs.at[s], dsts.at[s],
                       src_v, dst_v, rows_v, acc, gsem, ssem)

    return _sc_aggr


def _mlp_body(relu_out, split_out, z_ref, wa_ref, ba_ref, wb_ref, bb_ref,
              out_ref):
    h = jnp.dot(z_ref[0], wa_ref[:H, :], preferred_element_type=jnp.float32)
    h += jnp.dot(z_ref[1], wa_ref[H:, :], preferred_element_type=jnp.float32)
    h = jnp.maximum(h + ba_ref[...], 0.0)
    o = jnp.dot(h, wb_ref[...], preferred_element_type=jnp.float32) + bb_ref[...]
    if relu_out:
        o = jnp.maximum(o, 0.0)
    if split_out:
        out_ref[0] = o[:, :H]
        out_ref[1] = o[:, H:]
    else:
        out_ref[...] = o


def _mlp(z, wa, ba, wb, bb, relu_out, split_out):
    if split_out:
        blk = 2048
        grid = (NP // blk,)
        out_spec = pl.BlockSpec((2, blk, H), lambda i: (0, i, 0))
        out_shape = jax.ShapeDtypeStruct((2, NP, H), jnp.float32)
    else:
        blk = 2000
        grid = (N_NODES // blk,)
        out_spec = pl.BlockSpec((blk, D), lambda i: (i, 0))
        out_shape = jax.ShapeDtypeStruct((N_NODES, D), jnp.float32)
    return pl.pallas_call(
        functools.partial(_mlp_body, relu_out, split_out),
        grid=grid,
        in_specs=[
            pl.BlockSpec((2, blk, H), lambda i: (0, i, 0)),
            pl.BlockSpec((D, D), lambda i: (0, 0)),
            pl.BlockSpec((1, D), lambda i: (0, 0)),
            pl.BlockSpec((D, D), lambda i: (0, 0)),
            pl.BlockSpec((1, D), lambda i: (0, 0)),
        ],
        out_specs=out_spec,
        out_shape=out_shape,
    )(z, wa, ba.reshape(1, D), wb, bb.reshape(1, D))


def kernel(x, edge_index, W1a, b1a, W1b, b1b, W2a, b2a, W2b, b2b):
    pad_e = E_PAD - N_EDGES
    srcs = jnp.concatenate(
        [edge_index[0],
         jnp.zeros((pad_e,), jnp.int32)]).reshape(NS, NSTAGE, CH, K)
    dsts = jnp.concatenate(
        [edge_index[1],
         jnp.full((pad_e,), NP - 1, jnp.int32)]).reshape(NS, NSTAGE, CH, K)
    xp = jnp.pad(x, ((0, NP - N_NODES), (0, 0)))
    xs = jnp.stack([xp[:, :H], xp[:, H:]])
    sc_aggr = _make_sc_aggr()
    z1 = sc_aggr(xs, srcs, dsts)
    hs = _mlp(z1, W1a, b1a, W1b, b1b, relu_out=True, split_out=True)
    z2 = sc_aggr(hs, srcs, dsts)
    return _mlp(z2, W2a, b2a, W2b, b2b, relu_out=False, split_out=False)
